# baseline (speedup 1.0000x reference)
import functools

import jax
import jax.numpy as jnp
from jax import lax
from jax.experimental import pallas as pl
from jax.experimental.pallas import tpu as pltpu
from jax.experimental.pallas import tpu_sc as plsc

N = 10000
E = 320000
ES = 2 * E
LANE = 128
EROWS = 5120
EP = EROWS * LANE
P = 10240
ZROW = N
TKEY = N * N
ALPHA = 2.0
NC = 2
NS = 16
WROWS = EROWS // (NC * NS)
SROWS = EROWS // NS
BLK = 8
NROWS_PER_SUB = P // NS

_mesh = plsc.VectorSubcoreMesh(core_axis_name="c", subcore_axis_name="s")
_mesh1 = plsc.VectorSubcoreMesh(core_axis_name="c", subcore_axis_name="s",
                                num_cores=1)


@functools.partial(
    pl.kernel,
    out_type=jax.ShapeDtypeStruct((TKEY,), jnp.int32),
    mesh=_mesh,
    scratch_types=[
        pltpu.VMEM((BLK, LANE), jnp.int32),
        pltpu.VMEM((BLK, LANE), jnp.int32),
        pltpu.VMEM((BLK, LANE), jnp.int32),
        pltpu.VMEM((BLK, LANE), jnp.int32),
        pltpu.SemaphoreType.DMA,
        pltpu.SemaphoreType.DMA,
    ],
)
def _sc_dedup_scatter(keys_hbm, eids_hbm, t_hbm,
                      kbuf0, kbuf1, ebuf0, ebuf1, semi, sems):
    c = lax.axis_index("c")
    s = lax.axis_index("s")
    wid = c * NS + s
    row0 = wid * WROWS
    nblk = WROWS // BLK

    def base_of(b):
        return row0 + jnp.where(b >= nblk, b - nblk, b) * BLK

    def fire_idx(b, kbuf, ebuf):
        base = base_of(b)
        pltpu.async_copy(keys_hbm.at[pl.ds(base, BLK)], kbuf, semi)
        pltpu.async_copy(eids_hbm.at[pl.ds(base, BLK)], ebuf, semi)

    def wait_idx(kbuf, ebuf):
        pltpu.make_async_copy(keys_hbm.at[pl.ds(0, BLK)], kbuf, semi).wait()
        pltpu.make_async_copy(eids_hbm.at[pl.ds(0, BLK)], ebuf, semi).wait()

    def fire_scatters(kbuf, ebuf):
        for j in range(BLK):
            pltpu.async_copy(ebuf.at[j], t_hbm.at[kbuf.at[j]], sems)

    def wait_scatters(kbuf, ebuf):
        for j in range(BLK):
            pltpu.make_async_copy(ebuf.at[j], t_hbm.at[kbuf.at[j]],
                                  sems).wait()

    fire_idx(jnp.int32(0), kbuf0, ebuf0)
    wait_idx(kbuf0, ebuf0)
    fire_scatters(kbuf0, ebuf0)
    fire_idx(jnp.int32(1), kbuf1, ebuf1)
    wait_idx(kbuf1, ebuf1)
    fire_scatters(kbuf1, ebuf1)
    wait_scatters(kbuf0, ebuf0)
    fire_idx(jnp.int32(2), kbuf0, ebuf0)

    def group(g, carry):
        b0 = 2 * g
        wait_idx(kbuf0, ebuf0)
        fire_scatters(kbuf0, ebuf0)
        wait_scatters(kbuf1, ebuf1)
        fire_idx(b0 + 1, kbuf1, ebuf1)
        wait_idx(kbuf1, ebuf1)
        fire_scatters(kbuf1, ebuf1)
        wait_scatters(kbuf0, ebuf0)
        fire_idx(b0 + 2, kbuf0, ebuf0)
        return carry

    lax.fori_loop(1, nblk // 2, group, 0)
    wait_scatters(kbuf1, ebuf1)
    wait_idx(kbuf0, ebuf0)


@functools.partial(
    pl.kernel,
    out_type=(
        jax.ShapeDtypeStruct((EROWS, LANE), jnp.int32),
        jax.ShapeDtypeStruct((NC * N,), jnp.float32),
    ),
    mesh=_mesh,
    scratch_types=[
        pltpu.VMEM((BLK, LANE), jnp.int32),
        pltpu.VMEM((BLK, LANE), jnp.int32),
        pltpu.VMEM((BLK, LANE), jnp.int32),
        pltpu.VMEM((BLK, LANE), jnp.int32),
        pltpu.VMEM((BLK, LANE), jnp.int32),
        pltpu.VMEM((BLK, LANE), jnp.int32),
        pltpu.VMEM((BLK, LANE), jnp.float32),
        pltpu.VMEM((N,), jnp.float32),
        pltpu.VMEM_SHARED((N,), jnp.float32),
        pltpu.SemaphoreType.DMA,
    ],
)
def _sc_dedup_check(keys_hbm, eids_hbm, src_hbm, dst_hbm, t_hbm,
                    reff_hbm, deg_hbm,
                    kbuf, ebuf, sbuf, dbuf, tbuf, rbuf, obuf, dstage, deg_sp,
                    sem):
    c = lax.axis_index("c")
    s = lax.axis_index("s")
    wid = c * NS + s
    row0 = wid * WROWS

    @pl.when(s == 0)
    def _():
        def zrow(i, carry):
            dstage[pl.ds(i * 16, 16)] = jnp.zeros((16,), jnp.float32)
            return carry
        lax.fori_loop(0, N // 16, zrow, 0)
        pltpu.sync_copy(dstage, deg_sp)

    plsc.subcore_barrier()

    def block(b, carry):
        base = row0 + b * BLK
        pltpu.sync_copy(keys_hbm.at[pl.ds(base, BLK)], kbuf)
        pltpu.sync_copy(eids_hbm.at[pl.ds(base, BLK)], ebuf)
        pltpu.sync_copy(src_hbm.at[pl.ds(base, BLK)], sbuf)
        pltpu.sync_copy(dst_hbm.at[pl.ds(base, BLK)], dbuf)
        copies = []
        for j in range(BLK):
            copies.append(
                pltpu.async_copy(t_hbm.at[kbuf.at[j]], tbuf.at[j], sem))
        for cp in copies:
            cp.wait()
        for j in range(BLK):
            for i in range(LANE // 16):
                sl = pl.ds(i * 16, 16)
                tv = tbuf[j, sl]
                ev = ebuf[j, sl]
                sv = sbuf[j, sl]
                dv = dbuf[j, sl]
                valid = jnp.logical_and(tv == ev, sv != dv)
                zr = ZROW + jnp.bitwise_and(ev, 127)
                rbuf[j, sl] = jnp.where(valid, sv, zr)
                obuf[j, sl] = jnp.where(valid, 1.0, 0.0).astype(jnp.float32)
        pltpu.sync_copy(rbuf, reff_hbm.at[pl.ds(base, BLK)])
        for j in range(BLK):
            pltpu.sync_copy(obuf.at[j], deg_sp.at[dbuf.at[j]], add=True)
        return carry

    lax.fori_loop(0, WROWS // BLK, block, 0)
    plsc.subcore_barrier()

    @pl.when(s == 0)
    def _():
        pltpu.sync_copy(deg_sp, dstage)
        pltpu.sync_copy(dstage, deg_hbm.at[pl.ds(c * N, N)])


ABLK = 2
HALFN = P // NC
TRASH = HALFN
ACC_ROWS = HALFN + 512
NBLK = SROWS // ABLK


@functools.partial(
    pl.kernel,
    out_type=jax.ShapeDtypeStruct((P, 128), jnp.float32),
    mesh=_mesh,
    scratch_types=[
        pltpu.VMEM((ABLK, LANE), jnp.int32),
        pltpu.VMEM((ABLK, LANE), jnp.int32),
        pltpu.VMEM((ABLK, LANE), jnp.int32),
        pltpu.VMEM((ABLK, LANE), jnp.int32),
        pltpu.VMEM((ABLK, LANE), jnp.int32),
        pltpu.VMEM((ABLK, LANE), jnp.int32),
        pltpu.VMEM((ABLK * LANE, 128), jnp.float32),
        pltpu.VMEM((ABLK * LANE, 128), jnp.float32),
        pltpu.VMEM((LANE, 128), jnp.float32),
        pltpu.VMEM_SHARED((ACC_ROWS, 128), jnp.float32),
        pltpu.SemaphoreType.DMA,
        pltpu.SemaphoreType.DMA,
    ],
)
def _sc_aggregate(ytab_hbm, reff_hbm, dst_hbm, agg_hbm,
                  ridx0, ridx1, didx0, didx1, sidx0, sidx1,
                  gbuf0, gbuf1, stage, acc, semi, semg):
    c = lax.axis_index("c")
    s = lax.axis_index("s")

    def zrow(r, carry):
        for i in range(128 // 16):
            stage[r, pl.ds(i * 16, 16)] = jnp.zeros((16,), jnp.float32)
        return carry
    lax.fori_loop(0, LANE, zrow, 0)
    nzchunk = ACC_ROWS // LANE
    for t in range((nzchunk + NS - 1) // NS):
        ci = s + t * NS

        @pl.when(ci < nzchunk)
        def _():
            pltpu.sync_copy(stage, acc.at[pl.ds(ci * LANE, LANE)])
    plsc.subcore_barrier()

    row0 = s * SROWS
    lo = c * HALFN
    iota16 = lax.iota(jnp.int32, 16)

    def base_of(b):
        return row0 + jnp.where(b >= NBLK, b - NBLK, b) * ABLK

    def fire_idx(b, ridx, didx):
        base = base_of(b)
        pltpu.async_copy(reff_hbm.at[pl.ds(base, ABLK)], ridx, semi)
        pltpu.async_copy(dst_hbm.at[pl.ds(base, ABLK)], didx, semi)

    def wait_idx(ridx, didx):
        pltpu.make_async_copy(reff_hbm.at[pl.ds(0, ABLK)], ridx, semi).wait()
        pltpu.make_async_copy(dst_hbm.at[pl.ds(0, ABLK)], didx, semi).wait()

    def fire_gather(ridx, gbuf):
        for j in range(ABLK):
            pltpu.async_copy(ytab_hbm.at[ridx.at[j]],
                             gbuf.at[pl.ds(j * LANE, LANE)], semg)

    def wait_gather(gbuf):
        for j in range(ABLK):
            pltpu.make_async_copy(ytab_hbm.at[pl.ds(0, LANE)],
                                  gbuf.at[pl.ds(j * LANE, LANE)], semg).wait()

    def compute_sidx(b, didx, sidx):
        for j in range(ABLK):
            for i in range(LANE // 16):
                sl = pl.ds(i * 16, 16)
                dv = didx[j, sl]
                inr = jnp.logical_and(dv >= lo, dv < lo + HALFN)
                tr = TRASH + jnp.bitwise_and(
                    b * (ABLK * LANE) + j * LANE + i * 16 + iota16, 511)
                sidx[j, sl] = jnp.where(inr, dv - lo, tr)

    def scatter_add(sidx, gbuf):
        for j in range(ABLK):
            pltpu.sync_copy(gbuf.at[pl.ds(j * LANE, LANE)],
                            acc.at[sidx.at[j]], add=True)

    pltpu.sync_copy(reff_hbm.at[pl.ds(row0, ABLK)], ridx0)
    pltpu.sync_copy(dst_hbm.at[pl.ds(row0, ABLK)], didx0)
    fire_gather(ridx0, gbuf0)
    fire_idx(jnp.int32(1), ridx1, didx1)

    def group(g, carry):
        b0 = 2 * g
        compute_sidx(b0, didx0, sidx0)
        wait_gather(gbuf0)
        wait_idx(ridx1, didx1)
        fire_gather(ridx1, gbuf1)
        fire_idx(b0 + 2, ridx0, didx0)
        scatter_add(sidx0, gbuf0)
        compute_sidx(b0 + 1, didx1, sidx1)
        wait_gather(gbuf1)
        wait_idx(ridx0, didx0)
        fire_gather(ridx0, gbuf0)
        fire_idx(b0 + 3, ridx1, didx1)
        scatter_add(sidx1, gbuf1)
        return carry

    lax.fori_loop(0, NBLK // 2, group, 0)
    wait_gather(gbuf0)
    wait_idx(ridx1, didx1)

    plsc.subcore_barrier()
    wrow0 = s * (HALFN // NS)
    for t in range(2):
        r0 = wrow0 + t * LANE
        pltpu.sync_copy(acc.at[pl.ds(r0, LANE)], stage)
        pltpu.sync_copy(stage, agg_hbm.at[pl.ds(c * HALFN + r0, LANE)])
    r0 = wrow0 + 2 * LANE
    pltpu.sync_copy(acc.at[pl.ds(r0, 64)], stage.at[pl.ds(0, 64)])
    pltpu.sync_copy(stage.at[pl.ds(0, 64)],
                    agg_hbm.at[pl.ds(c * HALFN + r0, 64)])


def _tc_scale_matmul(h_pad, w, dis_pad):
    def body(h_ref, w_ref, dis_ref, out_ref):
        out_ref[...] = jnp.dot(h_ref[...], w_ref[...],
                               preferred_element_type=jnp.float32) * dis_ref[...]

    return pl.pallas_call(
        body,
        grid=(P // 512,),
        in_specs=[
            pl.BlockSpec((512, 128), lambda i: (i, 0)),
            pl.BlockSpec((128, 128), lambda i: (0, 0)),
            pl.BlockSpec((512, 1), lambda i: (i, 0)),
        ],
        out_specs=pl.BlockSpec((512, 128), lambda i: (i, 0)),
        out_shape=jax.ShapeDtypeStruct((P, 128), jnp.float32),
    )(h_pad, w, dis_pad)


def _tc_epilogue(aggp, ytab, dis_pad, b, g, lb, w_next):
    def body(agg_ref, ytab_ref, dis_ref, b_ref, g_ref, lb_ref, *rest):
        if w_next is not None:
            w_ref, out_ref = rest
        else:
            (out_ref,) = rest
        a = agg_ref[...]
        dis = dis_ref[...]
        o = dis * (a + ALPHA * ytab_ref[...]) + b_ref[...]
        h = jnp.maximum(o, 0.0)
        mu = jnp.mean(h, axis=-1, keepdims=True)
        var = jnp.mean((h - mu) ** 2, axis=-1, keepdims=True)
        hn = (h - mu) * lax.rsqrt(var + 1e-5) * g_ref[...] + lb_ref[...]
        if w_next is not None:
            out_ref[...] = jnp.dot(hn, w_ref[...],
                                   preferred_element_type=jnp.float32) * dis
        else:
            out_ref[...] = hn * dis

    in_specs = [
        pl.BlockSpec((512, 128), lambda i: (i, 0)),
        pl.BlockSpec((512, 128), lambda i: (i, 0)),
        pl.BlockSpec((512, 1), lambda i: (i, 0)),
        pl.BlockSpec((1, 128), lambda i: (0, 0)),
        pl.BlockSpec((1, 128), lambda i: (0, 0)),
        pl.BlockSpec((1, 128), lambda i: (0, 0)),
    ]
    args = [aggp, ytab, dis_pad, b, g, lb]
    if w_next is not None:
        in_specs.append(pl.BlockSpec((128, 128), lambda i: (0, 0)))
        args.append(w_next)
    return pl.pallas_call(
        body,
        grid=(P // 512,),
        in_specs=in_specs,
        out_specs=pl.BlockSpec((512, 128), lambda i: (i, 0)),
        out_shape=jax.ShapeDtypeStruct((P, 128), jnp.float32),
    )(*args)


def _tc_final(aggp, ttab, dis_pad, w2, b2):
    def body(agg_ref, t_ref, dis_ref, w_ref, b_ref, emb_ref, logp_ref):
        a = agg_ref[...]
        pre = dis_ref[...] * (a + ALPHA * t_ref[...])
        emb = jnp.dot(pre, w_ref[...],
                      preferred_element_type=jnp.float32) + b_ref[...]
        m = jnp.max(emb, axis=-1, keepdims=True)
        ex = jnp.exp(emb - m)
        lse = jnp.log(jnp.sum(ex, axis=-1, keepdims=True)) + m
        emb_ref[...] = emb
        logp_ref[...] = emb - lse

    return pl.pallas_call(
        body,
        grid=(25,),
        in_specs=[
            pl.BlockSpec((400, 128), lambda i: (i, 0)),
            pl.BlockSpec((400, 128), lambda i: (i, 0)),
            pl.BlockSpec((400, 1), lambda i: (i, 0)),
            pl.BlockSpec((128, 40), lambda i: (0, 0)),
            pl.BlockSpec((1, 40), lambda i: (0, 0)),
        ],
        out_specs=[
            pl.BlockSpec((400, 40), lambda i: (i, 0)),
            pl.BlockSpec((400, 40), lambda i: (i, 0)),
        ],
        out_shape=[
            jax.ShapeDtypeStruct((N, 40), jnp.float32),
            jax.ShapeDtypeStruct((N, 40), jnp.float32),
        ],
    )(aggp, ttab, dis_pad, w2, b2)


def kernel(x, edge_index, W0, b0, W1, b1, W2, b2, ln0_g, ln0_b, ln1_g, ln1_b):
    src = edge_index[0].astype(jnp.int32)
    dst = edge_index[1].astype(jnp.int32)
    pad_len = EP - ES
    pad_node = (jnp.arange(pad_len, dtype=jnp.int32)) % N
    s_all = jnp.concatenate([src, dst, pad_node])
    d_all = jnp.concatenate([dst, src, pad_node])
    keys = s_all * N + d_all
    eids = jnp.arange(EP, dtype=jnp.int32)

    keys2 = keys.reshape(EROWS, LANE)
    eids2 = eids.reshape(EROWS, LANE)
    s2 = s_all.reshape(EROWS, LANE)
    d2 = d_all.reshape(EROWS, LANE)

    t_tab = _sc_dedup_scatter(keys2, eids2)
    reff2, deg_p = _sc_dedup_check(keys2, eids2, s2, d2, t_tab)

    deg = deg_p[:N] + deg_p[N:] + jnp.float32(ALPHA)
    dis = deg ** -0.5
    dis_pad = jnp.pad(dis, (0, P - N)).reshape(P, 1)

    x_pad = jnp.pad(x, ((0, P - N), (0, 0)))

    y0 = _tc_scale_matmul(x_pad, W0, dis_pad)
    agg0 = _sc_aggregate(y0, reff2, d2)
    y1 = _tc_epilogue(agg0, y0, dis_pad, b0.reshape(1, 128),
                      ln0_g.reshape(1, 128), ln0_b.reshape(1, 128), W1)
    agg1 = _sc_aggregate(y1, reff2, d2)
    t2 = _tc_epilogue(agg1, y1, dis_pad, b1.reshape(1, 128),
                      ln1_g.reshape(1, 128), ln1_b.reshape(1, 128), None)
    agg2 = _sc_aggregate(t2, reff2, d2)
    emb, logp = _tc_final(agg2, t2, dis_pad, W2, b2.reshape(1, 40))
    return (emb, logp)


# speedup vs baseline: 1.0016x; 1.0016x over previous
import functools

import jax
import jax.numpy as jnp
from jax import lax
from jax.experimental import pallas as pl
from jax.experimental.pallas import tpu as pltpu
from jax.experimental.pallas import tpu_sc as plsc

N = 10000
E = 320000
ES = 2 * E
LANE = 128
EROWS = 5120
EP = EROWS * LANE
P = 10240
ZROW = N
TKEY = N * N
ALPHA = 2.0
NC = 2
NS = 16
WROWS = EROWS // (NC * NS)
SROWS = EROWS // NS
BLK = 8
NROWS_PER_SUB = P // NS

_mesh = plsc.VectorSubcoreMesh(core_axis_name="c", subcore_axis_name="s")
_mesh1 = plsc.VectorSubcoreMesh(core_axis_name="c", subcore_axis_name="s",
                                num_cores=1)


@functools.partial(
    pl.kernel,
    out_type=jax.ShapeDtypeStruct((TKEY,), jnp.int32),
    mesh=_mesh,
    scratch_types=[
        pltpu.VMEM((BLK, LANE), jnp.int32),
        pltpu.VMEM((BLK, LANE), jnp.int32),
        pltpu.VMEM((BLK, LANE), jnp.int32),
        pltpu.VMEM((BLK, LANE), jnp.int32),
        pltpu.SemaphoreType.DMA,
        pltpu.SemaphoreType.DMA,
    ],
)
def _sc_dedup_scatter(keys_hbm, eids_hbm, t_hbm,
                      kbuf0, kbuf1, ebuf0, ebuf1, semi, sems):
    c = lax.axis_index("c")
    s = lax.axis_index("s")
    wid = c * NS + s
    row0 = wid * WROWS
    nblk = WROWS // BLK

    def base_of(b):
        return row0 + jnp.where(b >= nblk, b - nblk, b) * BLK

    def fire_idx(b, kbuf, ebuf):
        base = base_of(b)
        pltpu.async_copy(keys_hbm.at[pl.ds(base, BLK)], kbuf, semi)
        pltpu.async_copy(eids_hbm.at[pl.ds(base, BLK)], ebuf, semi)

    def wait_idx(kbuf, ebuf):
        pltpu.make_async_copy(keys_hbm.at[pl.ds(0, BLK)], kbuf, semi).wait()
        pltpu.make_async_copy(eids_hbm.at[pl.ds(0, BLK)], ebuf, semi).wait()

    def fire_scatters(kbuf, ebuf):
        for j in range(BLK):
            pltpu.async_copy(ebuf.at[j], t_hbm.at[kbuf.at[j]], sems)

    def wait_scatters(kbuf, ebuf):
        for j in range(BLK):
            pltpu.make_async_copy(ebuf.at[j], t_hbm.at[kbuf.at[j]],
                                  sems).wait()

    fire_idx(jnp.int32(0), kbuf0, ebuf0)
    wait_idx(kbuf0, ebuf0)
    fire_scatters(kbuf0, ebuf0)
    fire_idx(jnp.int32(1), kbuf1, ebuf1)
    wait_idx(kbuf1, ebuf1)
    fire_scatters(kbuf1, ebuf1)
    wait_scatters(kbuf0, ebuf0)
    fire_idx(jnp.int32(2), kbuf0, ebuf0)

    def group(g, carry):
        b0 = 2 * g
        wait_idx(kbuf0, ebuf0)
        fire_scatters(kbuf0, ebuf0)
        wait_scatters(kbuf1, ebuf1)
        fire_idx(b0 + 1, kbuf1, ebuf1)
        wait_idx(kbuf1, ebuf1)
        fire_scatters(kbuf1, ebuf1)
        wait_scatters(kbuf0, ebuf0)
        fire_idx(b0 + 2, kbuf0, ebuf0)
        return carry

    lax.fori_loop(1, nblk // 2, group, 0)
    wait_scatters(kbuf1, ebuf1)
    wait_idx(kbuf0, ebuf0)


@functools.partial(
    pl.kernel,
    out_type=(
        jax.ShapeDtypeStruct((EROWS, LANE), jnp.int32),
        jax.ShapeDtypeStruct((NC * N,), jnp.float32),
    ),
    mesh=_mesh,
    scratch_types=[
        pltpu.VMEM((BLK, LANE), jnp.int32),
        pltpu.VMEM((BLK, LANE), jnp.int32),
        pltpu.VMEM((BLK, LANE), jnp.int32),
        pltpu.VMEM((BLK, LANE), jnp.int32),
        pltpu.VMEM((BLK, LANE), jnp.int32),
        pltpu.VMEM((BLK, LANE), jnp.int32),
        pltpu.VMEM((BLK, LANE), jnp.float32),
        pltpu.VMEM((N,), jnp.float32),
        pltpu.VMEM_SHARED((N,), jnp.float32),
        pltpu.SemaphoreType.DMA,
    ],
)
def _sc_dedup_check(keys_hbm, eids_hbm, src_hbm, dst_hbm, t_hbm,
                    reff_hbm, deg_hbm,
                    kbuf, ebuf, sbuf, dbuf, tbuf, rbuf, obuf, dstage, deg_sp,
                    sem):
    c = lax.axis_index("c")
    s = lax.axis_index("s")
    wid = c * NS + s
    row0 = wid * WROWS

    @pl.when(s == 0)
    def _():
        def zrow(i, carry):
            dstage[pl.ds(i * 16, 16)] = jnp.zeros((16,), jnp.float32)
            return carry
        lax.fori_loop(0, N // 16, zrow, 0)
        pltpu.sync_copy(dstage, deg_sp)

    plsc.subcore_barrier()

    def block(b, carry):
        base = row0 + b * BLK
        pltpu.sync_copy(keys_hbm.at[pl.ds(base, BLK)], kbuf)
        pltpu.sync_copy(eids_hbm.at[pl.ds(base, BLK)], ebuf)
        pltpu.sync_copy(src_hbm.at[pl.ds(base, BLK)], sbuf)
        pltpu.sync_copy(dst_hbm.at[pl.ds(base, BLK)], dbuf)
        copies = []
        for j in range(BLK):
            copies.append(
                pltpu.async_copy(t_hbm.at[kbuf.at[j]], tbuf.at[j], sem))
        for cp in copies:
            cp.wait()
        for j in range(BLK):
            for i in range(LANE // 16):
                sl = pl.ds(i * 16, 16)
                tv = tbuf[j, sl]
                ev = ebuf[j, sl]
                sv = sbuf[j, sl]
                dv = dbuf[j, sl]
                valid = jnp.logical_and(tv == ev, sv != dv)
                zr = ZROW + jnp.bitwise_and(ev, 127)
                rbuf[j, sl] = jnp.where(valid, sv, zr)
                obuf[j, sl] = jnp.where(valid, 1.0, 0.0).astype(jnp.float32)
        pltpu.sync_copy(rbuf, reff_hbm.at[pl.ds(base, BLK)])
        for j in range(BLK):
            pltpu.sync_copy(obuf.at[j], deg_sp.at[dbuf.at[j]], add=True)
        return carry

    lax.fori_loop(0, WROWS // BLK, block, 0)
    plsc.subcore_barrier()

    @pl.when(s == 0)
    def _():
        pltpu.sync_copy(deg_sp, dstage)
        pltpu.sync_copy(dstage, deg_hbm.at[pl.ds(c * N, N)])


ABLK = 2
HALFN = P // NC
TRASH = HALFN
ACC_ROWS = HALFN + 512
NBLK = SROWS // ABLK


@functools.partial(
    pl.kernel,
    out_type=jax.ShapeDtypeStruct((P, 128), jnp.float32),
    mesh=_mesh,
    scratch_types=[
        pltpu.VMEM((ABLK, LANE), jnp.int32),
        pltpu.VMEM((ABLK, LANE), jnp.int32),
        pltpu.VMEM((ABLK, LANE), jnp.int32),
        pltpu.VMEM((ABLK, LANE), jnp.int32),
        pltpu.VMEM((ABLK, LANE), jnp.int32),
        pltpu.VMEM((ABLK, LANE), jnp.int32),
        pltpu.VMEM((ABLK * LANE, 128), jnp.float32),
        pltpu.VMEM((ABLK * LANE, 128), jnp.float32),
        pltpu.VMEM((LANE, 128), jnp.float32),
        pltpu.VMEM_SHARED((ACC_ROWS, 128), jnp.float32),
        pltpu.SemaphoreType.DMA,
        pltpu.SemaphoreType.DMA,
        pltpu.SemaphoreType.DMA,
    ],
)
def _sc_aggregate(ytab_hbm, reff_hbm, dst_hbm, agg_hbm,
                  ridx0, ridx1, didx0, didx1, sidx0, sidx1,
                  gbuf0, gbuf1, stage, acc, semi, semg, sems):
    c = lax.axis_index("c")
    s = lax.axis_index("s")

    def zrow(r, carry):
        for i in range(128 // 16):
            stage[r, pl.ds(i * 16, 16)] = jnp.zeros((16,), jnp.float32)
        return carry
    lax.fori_loop(0, LANE, zrow, 0)
    nzchunk = ACC_ROWS // LANE
    for t in range((nzchunk + NS - 1) // NS):
        ci = s + t * NS

        @pl.when(ci < nzchunk)
        def _():
            pltpu.sync_copy(stage, acc.at[pl.ds(ci * LANE, LANE)])
    plsc.subcore_barrier()

    row0 = s * SROWS
    lo = c * HALFN
    iota16 = lax.iota(jnp.int32, 16)

    def base_of(b):
        return row0 + jnp.where(b >= NBLK, b - NBLK, b) * ABLK

    def fire_idx(b, ridx, didx):
        base = base_of(b)
        pltpu.async_copy(reff_hbm.at[pl.ds(base, ABLK)], ridx, semi)
        pltpu.async_copy(dst_hbm.at[pl.ds(base, ABLK)], didx, semi)

    def wait_idx(ridx, didx):
        pltpu.make_async_copy(reff_hbm.at[pl.ds(0, ABLK)], ridx, semi).wait()
        pltpu.make_async_copy(dst_hbm.at[pl.ds(0, ABLK)], didx, semi).wait()

    def fire_gather(ridx, gbuf):
        for j in range(ABLK):
            pltpu.async_copy(ytab_hbm.at[ridx.at[j]],
                             gbuf.at[pl.ds(j * LANE, LANE)], semg)

    def wait_gather(gbuf):
        for j in range(ABLK):
            pltpu.make_async_copy(ytab_hbm.at[pl.ds(0, LANE)],
                                  gbuf.at[pl.ds(j * LANE, LANE)], semg).wait()

    def compute_sidx(b, didx, sidx):
        for j in range(ABLK):
            for i in range(LANE // 16):
                sl = pl.ds(i * 16, 16)
                dv = didx[j, sl]
                inr = jnp.logical_and(dv >= lo, dv < lo + HALFN)
                tr = TRASH + jnp.bitwise_and(
                    b * (ABLK * LANE) + j * LANE + i * 16 + iota16, 511)
                sidx[j, sl] = jnp.where(inr, dv - lo, tr)

    def fire_scat(sidx, gbuf):
        for j in range(ABLK):
            pltpu.async_copy(gbuf.at[pl.ds(j * LANE, LANE)],
                             acc.at[sidx.at[j]], sems, add=True)

    def wait_scat(sidx, gbuf):
        for j in range(ABLK):
            pltpu.make_async_copy(gbuf.at[pl.ds(j * LANE, LANE)],
                                  acc.at[sidx.at[j]], sems).wait()

    pltpu.sync_copy(reff_hbm.at[pl.ds(row0, ABLK)], ridx0)
    pltpu.sync_copy(dst_hbm.at[pl.ds(row0, ABLK)], didx0)
    fire_gather(ridx0, gbuf0)
    fire_idx(jnp.int32(1), ridx1, didx1)

    compute_sidx(jnp.int32(0), didx0, sidx0)
    wait_gather(gbuf0)
    wait_idx(ridx1, didx1)
    fire_gather(ridx1, gbuf1)
    fire_idx(jnp.int32(2), ridx0, didx0)
    fire_scat(sidx0, gbuf0)
    compute_sidx(jnp.int32(1), didx1, sidx1)
    wait_gather(gbuf1)
    wait_idx(ridx0, didx0)
    wait_scat(sidx0, gbuf0)
    fire_gather(ridx0, gbuf0)
    fire_idx(jnp.int32(3), ridx1, didx1)
    fire_scat(sidx1, gbuf1)

    def group(g, carry):
        b0 = 2 * g
        compute_sidx(b0, didx0, sidx0)
        wait_gather(gbuf0)
        wait_idx(ridx1, didx1)
        wait_scat(sidx1, gbuf1)
        fire_gather(ridx1, gbuf1)
        fire_idx(b0 + 2, ridx0, didx0)
        fire_scat(sidx0, gbuf0)
        compute_sidx(b0 + 1, didx1, sidx1)
        wait_gather(gbuf1)
        wait_idx(ridx0, didx0)
        wait_scat(sidx0, gbuf0)
        fire_gather(ridx0, gbuf0)
        fire_idx(b0 + 3, ridx1, didx1)
        fire_scat(sidx1, gbuf1)
        return carry

    lax.fori_loop(1, NBLK // 2, group, 0)
    wait_gather(gbuf0)
    wait_idx(ridx1, didx1)
    wait_scat(sidx1, gbuf1)

    plsc.subcore_barrier()
    wrow0 = s * (HALFN // NS)
    for t in range(2):
        r0 = wrow0 + t * LANE
        pltpu.sync_copy(acc.at[pl.ds(r0, LANE)], stage)
        pltpu.sync_copy(stage, agg_hbm.at[pl.ds(c * HALFN + r0, LANE)])
    r0 = wrow0 + 2 * LANE
    pltpu.sync_copy(acc.at[pl.ds(r0, 64)], stage.at[pl.ds(0, 64)])
    pltpu.sync_copy(stage.at[pl.ds(0, 64)],
                    agg_hbm.at[pl.ds(c * HALFN + r0, 64)])


def _tc_scale_matmul(h_pad, w, dis_pad):
    def body(h_ref, w_ref, dis_ref, out_ref):
        out_ref[...] = jnp.dot(h_ref[...], w_ref[...],
                               preferred_element_type=jnp.float32) * dis_ref[...]

    return pl.pallas_call(
        body,
        grid=(P // 512,),
        in_specs=[
            pl.BlockSpec((512, 128), lambda i: (i, 0)),
            pl.BlockSpec((128, 128), lambda i: (0, 0)),
            pl.BlockSpec((512, 1), lambda i: (i, 0)),
        ],
        out_specs=pl.BlockSpec((512, 128), lambda i: (i, 0)),
        out_shape=jax.ShapeDtypeStruct((P, 128), jnp.float32),
    )(h_pad, w, dis_pad)


def _tc_epilogue(aggp, ytab, dis_pad, b, g, lb, w_next):
    def body(agg_ref, ytab_ref, dis_ref, b_ref, g_ref, lb_ref, *rest):
        if w_next is not None:
            w_ref, out_ref = rest
        else:
            (out_ref,) = rest
        a = agg_ref[...]
        dis = dis_ref[...]
        o = dis * (a + ALPHA * ytab_ref[...]) + b_ref[...]
        h = jnp.maximum(o, 0.0)
        mu = jnp.mean(h, axis=-1, keepdims=True)
        var = jnp.mean((h - mu) ** 2, axis=-1, keepdims=True)
        hn = (h - mu) * lax.rsqrt(var + 1e-5) * g_ref[...] + lb_ref[...]
        if w_next is not None:
            out_ref[...] = jnp.dot(hn, w_ref[...],
                                   preferred_element_type=jnp.float32) * dis
        else:
            out_ref[...] = hn * dis

    in_specs = [
        pl.BlockSpec((512, 128), lambda i: (i, 0)),
        pl.BlockSpec((512, 128), lambda i: (i, 0)),
        pl.BlockSpec((512, 1), lambda i: (i, 0)),
        pl.BlockSpec((1, 128), lambda i: (0, 0)),
        pl.BlockSpec((1, 128), lambda i: (0, 0)),
        pl.BlockSpec((1, 128), lambda i: (0, 0)),
    ]
    args = [aggp, ytab, dis_pad, b, g, lb]
    if w_next is not None:
        in_specs.append(pl.BlockSpec((128, 128), lambda i: (0, 0)))
        args.append(w_next)
    return pl.pallas_call(
        body,
        grid=(P // 512,),
        in_specs=in_specs,
        out_specs=pl.BlockSpec((512, 128), lambda i: (i, 0)),
        out_shape=jax.ShapeDtypeStruct((P, 128), jnp.float32),
    )(*args)


def _tc_final(aggp, ttab, dis_pad, w2, b2):
    def body(agg_ref, t_ref, dis_ref, w_ref, b_ref, emb_ref, logp_ref):
        a = agg_ref[...]
        pre = dis_ref[...] * (a + ALPHA * t_ref[...])
        emb = jnp.dot(pre, w_ref[...],
                      preferred_element_type=jnp.float32) + b_ref[...]
        m = jnp.max(emb, axis=-1, keepdims=True)
        ex = jnp.exp(emb - m)
        lse = jnp.log(jnp.sum(ex, axis=-1, keepdims=True)) + m
        emb_ref[...] = emb
        logp_ref[...] = emb - lse

    return pl.pallas_call(
        body,
        grid=(25,),
        in_specs=[
            pl.BlockSpec((400, 128), lambda i: (i, 0)),
            pl.BlockSpec((400, 128), lambda i: (i, 0)),
            pl.BlockSpec((400, 1), lambda i: (i, 0)),
            pl.BlockSpec((128, 40), lambda i: (0, 0)),
            pl.BlockSpec((1, 40), lambda i: (0, 0)),
        ],
        out_specs=[
            pl.BlockSpec((400, 40), lambda i: (i, 0)),
            pl.BlockSpec((400, 40), lambda i: (i, 0)),
        ],
        out_shape=[
            jax.ShapeDtypeStruct((N, 40), jnp.float32),
            jax.ShapeDtypeStruct((N, 40), jnp.float32),
        ],
    )(aggp, ttab, dis_pad, w2, b2)


def kernel(x, edge_index, W0, b0, W1, b1, W2, b2, ln0_g, ln0_b, ln1_g, ln1_b):
    src = edge_index[0].astype(jnp.int32)
    dst = edge_index[1].astype(jnp.int32)
    pad_len = EP - ES
    pad_node = (jnp.arange(pad_len, dtype=jnp.int32)) % N
    s_all = jnp.concatenate([src, dst, pad_node])
    d_all = jnp.concatenate([dst, src, pad_node])
    keys = s_all * N + d_all
    eids = jnp.arange(EP, dtype=jnp.int32)

    keys2 = keys.reshape(EROWS, LANE)
    eids2 = eids.reshape(EROWS, LANE)
    s2 = s_all.reshape(EROWS, LANE)
    d2 = d_all.reshape(EROWS, LANE)

    t_tab = _sc_dedup_scatter(keys2, eids2)
    reff2, deg_p = _sc_dedup_check(keys2, eids2, s2, d2, t_tab)

    deg = deg_p[:N] + deg_p[N:] + jnp.float32(ALPHA)
    dis = deg ** -0.5
    dis_pad = jnp.pad(dis, (0, P - N)).reshape(P, 1)

    x_pad = jnp.pad(x, ((0, P - N), (0, 0)))

    y0 = _tc_scale_matmul(x_pad, W0, dis_pad)
    agg0 = _sc_aggregate(y0, reff2, d2)
    y1 = _tc_epilogue(agg0, y0, dis_pad, b0.reshape(1, 128),
                      ln0_g.reshape(1, 128), ln0_b.reshape(1, 128), W1)
    agg1 = _sc_aggregate(y1, reff2, d2)
    t2 = _tc_epilogue(agg1, y1, dis_pad, b1.reshape(1, 128),
                      ln1_g.reshape(1, 128), ln1_b.reshape(1, 128), None)
    agg2 = _sc_aggregate(t2, reff2, d2)
    emb, logp = _tc_final(agg2, t2, dis_pad, W2, b2.reshape(1, 40))
    return (emb, logp)


# speedup vs baseline: 1.0164x; 1.0148x over previous
import functools

import jax
import jax.numpy as jnp
from jax import lax
from jax.experimental import pallas as pl
from jax.experimental.pallas import tpu as pltpu
from jax.experimental.pallas import tpu_sc as plsc

N = 10000
E = 320000
ES = 2 * E
LANE = 128
EROWS = 5120
EP = EROWS * LANE
P = 10240
ZROW = N
TKEY = N * N
ALPHA = 2.0
NC = 2
NS = 16
WROWS = EROWS // (NC * NS)
SROWS = EROWS // NS
BLK = 8
NROWS_PER_SUB = P // NS

_mesh = plsc.VectorSubcoreMesh(core_axis_name="c", subcore_axis_name="s")
_mesh1 = plsc.VectorSubcoreMesh(core_axis_name="c", subcore_axis_name="s",
                                num_cores=1)


@functools.partial(
    pl.kernel,
    out_type=jax.ShapeDtypeStruct((TKEY,), jnp.int32),
    mesh=_mesh,
    scratch_types=[
        pltpu.VMEM((BLK, LANE), jnp.int32),
        pltpu.VMEM((BLK, LANE), jnp.int32),
        pltpu.VMEM((BLK, LANE), jnp.int32),
        pltpu.VMEM((BLK, LANE), jnp.int32),
        pltpu.SemaphoreType.DMA,
        pltpu.SemaphoreType.DMA,
    ],
)
def _sc_dedup_scatter(keys_hbm, t_hbm,
                      kbuf0, kbuf1, ebuf0, ebuf1, semi, sems):
    c = lax.axis_index("c")
    s = lax.axis_index("s")
    wid = c * NS + s
    row0 = wid * WROWS
    nblk = WROWS // BLK
    iota16 = lax.iota(jnp.int32, 16)

    def base_of(b):
        return row0 + jnp.where(b >= nblk, b - nblk, b) * BLK

    def fire_idx(b, kbuf, ebuf):
        base = base_of(b)
        pltpu.async_copy(keys_hbm.at[pl.ds(base, BLK)], kbuf, semi)
        for j in range(BLK):
            for i in range(LANE // 16):
                ebuf[j, pl.ds(i * 16, 16)] = (base + j) * LANE + i * 16 + iota16

    def wait_idx(kbuf, ebuf):
        pltpu.make_async_copy(keys_hbm.at[pl.ds(0, BLK)], kbuf, semi).wait()

    def fire_scatters(kbuf, ebuf):
        for j in range(BLK):
            pltpu.async_copy(ebuf.at[j], t_hbm.at[kbuf.at[j]], sems)

    def wait_scatters(kbuf, ebuf):
        for j in range(BLK):
            pltpu.make_async_copy(ebuf.at[j], t_hbm.at[kbuf.at[j]],
                                  sems).wait()

    fire_idx(jnp.int32(0), kbuf0, ebuf0)
    wait_idx(kbuf0, ebuf0)
    fire_scatters(kbuf0, ebuf0)
    fire_idx(jnp.int32(1), kbuf1, ebuf1)
    wait_idx(kbuf1, ebuf1)
    fire_scatters(kbuf1, ebuf1)
    wait_scatters(kbuf0, ebuf0)
    fire_idx(jnp.int32(2), kbuf0, ebuf0)

    def group(g, carry):
        b0 = 2 * g
        wait_idx(kbuf0, ebuf0)
        fire_scatters(kbuf0, ebuf0)
        wait_scatters(kbuf1, ebuf1)
        fire_idx(b0 + 1, kbuf1, ebuf1)
        wait_idx(kbuf1, ebuf1)
        fire_scatters(kbuf1, ebuf1)
        wait_scatters(kbuf0, ebuf0)
        fire_idx(b0 + 2, kbuf0, ebuf0)
        return carry

    lax.fori_loop(1, nblk // 2, group, 0)
    wait_scatters(kbuf1, ebuf1)
    wait_idx(kbuf0, ebuf0)


@functools.partial(
    pl.kernel,
    out_type=(
        jax.ShapeDtypeStruct((EROWS, LANE), jnp.int32),
        jax.ShapeDtypeStruct((NC * N,), jnp.float32),
    ),
    mesh=_mesh,
    scratch_types=[
        pltpu.VMEM((BLK, LANE), jnp.int32),
        pltpu.VMEM((BLK, LANE), jnp.int32),
        pltpu.VMEM((BLK, LANE), jnp.int32),
        pltpu.VMEM((BLK, LANE), jnp.int32),
        pltpu.VMEM((BLK, LANE), jnp.int32),
        pltpu.VMEM((BLK, LANE), jnp.int32),
        pltpu.VMEM((BLK, LANE), jnp.float32),
        pltpu.VMEM((N,), jnp.float32),
        pltpu.VMEM_SHARED((N,), jnp.float32),
        pltpu.SemaphoreType.DMA,
    ],
)
def _sc_dedup_check(keys_hbm, src_hbm, dst_hbm, t_hbm,
                    reff_hbm, deg_hbm,
                    kbuf, ebuf, sbuf, dbuf, tbuf, rbuf, obuf, dstage, deg_sp,
                    sem):
    c = lax.axis_index("c")
    s = lax.axis_index("s")
    wid = c * NS + s
    row0 = wid * WROWS
    iota16 = lax.iota(jnp.int32, 16)

    @pl.when(s == 0)
    def _():
        def zrow(i, carry):
            dstage[pl.ds(i * 16, 16)] = jnp.zeros((16,), jnp.float32)
            return carry
        lax.fori_loop(0, N // 16, zrow, 0)
        pltpu.sync_copy(dstage, deg_sp)

    plsc.subcore_barrier()

    def block(b, carry):
        base = row0 + b * BLK
        loads = [
            pltpu.async_copy(keys_hbm.at[pl.ds(base, BLK)], kbuf, sem),
            pltpu.async_copy(src_hbm.at[pl.ds(base, BLK)], sbuf, sem),
            pltpu.async_copy(dst_hbm.at[pl.ds(base, BLK)], dbuf, sem),
        ]
        for cp in loads:
            cp.wait()
        copies = []
        for j in range(BLK):
            copies.append(
                pltpu.async_copy(t_hbm.at[kbuf.at[j]], tbuf.at[j], sem))
        for cp in copies:
            cp.wait()
        for j in range(BLK):
            for i in range(LANE // 16):
                sl = pl.ds(i * 16, 16)
                tv = tbuf[j, sl]
                ev = (base + j) * LANE + i * 16 + iota16
                sv = sbuf[j, sl]
                dv = dbuf[j, sl]
                valid = jnp.logical_and(tv == ev, sv != dv)
                zr = ZROW + jnp.bitwise_and(ev, 127)
                rbuf[j, sl] = jnp.where(valid, sv, zr)
                obuf[j, sl] = jnp.where(valid, 1.0, 0.0).astype(jnp.float32)
        pltpu.sync_copy(rbuf, reff_hbm.at[pl.ds(base, BLK)])
        for j in range(BLK):
            pltpu.sync_copy(obuf.at[j], deg_sp.at[dbuf.at[j]], add=True)
        return carry

    lax.fori_loop(0, WROWS // BLK, block, 0)
    plsc.subcore_barrier()

    @pl.when(s == 0)
    def _():
        pltpu.sync_copy(deg_sp, dstage)
        pltpu.sync_copy(dstage, deg_hbm.at[pl.ds(c * N, N)])


ABLK = 2
HALFN = P // NC
TRASH = HALFN
ACC_ROWS = HALFN + 512
NBLK = SROWS // ABLK


@functools.partial(
    pl.kernel,
    out_type=jax.ShapeDtypeStruct((P, 128), jnp.float32),
    mesh=_mesh,
    scratch_types=[
        pltpu.VMEM((ABLK, LANE), jnp.int32),
        pltpu.VMEM((ABLK, LANE), jnp.int32),
        pltpu.VMEM((ABLK, LANE), jnp.int32),
        pltpu.VMEM((ABLK, LANE), jnp.int32),
        pltpu.VMEM((ABLK, LANE), jnp.int32),
        pltpu.VMEM((ABLK, LANE), jnp.int32),
        pltpu.VMEM((ABLK * LANE, 128), jnp.float32),
        pltpu.VMEM((ABLK * LANE, 128), jnp.float32),
        pltpu.VMEM((LANE, 128), jnp.float32),
        pltpu.VMEM_SHARED((ACC_ROWS, 128), jnp.float32),
        pltpu.SemaphoreType.DMA,
        pltpu.SemaphoreType.DMA,
        pltpu.SemaphoreType.DMA,
    ],
)
def _sc_aggregate(ytab_hbm, reff_hbm, dst_hbm, agg_hbm,
                  ridx0, ridx1, didx0, didx1, sidx0, sidx1,
                  gbuf0, gbuf1, stage, acc, semi, semg, sems):
    c = lax.axis_index("c")
    s = lax.axis_index("s")

    def zrow(r, carry):
        for i in range(128 // 16):
            stage[r, pl.ds(i * 16, 16)] = jnp.zeros((16,), jnp.float32)
        return carry
    lax.fori_loop(0, LANE, zrow, 0)
    nzchunk = ACC_ROWS // LANE
    for t in range((nzchunk + NS - 1) // NS):
        ci = s + t * NS

        @pl.when(ci < nzchunk)
        def _():
            pltpu.sync_copy(stage, acc.at[pl.ds(ci * LANE, LANE)])
    plsc.subcore_barrier()

    row0 = s * SROWS
    lo = c * HALFN
    iota16 = lax.iota(jnp.int32, 16)

    def base_of(b):
        return row0 + jnp.where(b >= NBLK, b - NBLK, b) * ABLK

    def fire_idx(b, ridx, didx):
        base = base_of(b)
        pltpu.async_copy(reff_hbm.at[pl.ds(base, ABLK)], ridx, semi)
        pltpu.async_copy(dst_hbm.at[pl.ds(base, ABLK)], didx, semi)

    def wait_idx(ridx, didx):
        pltpu.make_async_copy(reff_hbm.at[pl.ds(0, ABLK)], ridx, semi).wait()
        pltpu.make_async_copy(dst_hbm.at[pl.ds(0, ABLK)], didx, semi).wait()

    def fire_gather(ridx, gbuf):
        for j in range(ABLK):
            pltpu.async_copy(ytab_hbm.at[ridx.at[j]],
                             gbuf.at[pl.ds(j * LANE, LANE)], semg)

    def wait_gather(gbuf):
        for j in range(ABLK):
            pltpu.make_async_copy(ytab_hbm.at[pl.ds(0, LANE)],
                                  gbuf.at[pl.ds(j * LANE, LANE)], semg).wait()

    def compute_sidx(b, didx, sidx):
        for j in range(ABLK):
            for i in range(LANE // 16):
                sl = pl.ds(i * 16, 16)
                dv = didx[j, sl]
                inr = jnp.logical_and(dv >= lo, dv < lo + HALFN)
                tr = TRASH + jnp.bitwise_and(
                    b * (ABLK * LANE) + j * LANE + i * 16 + iota16, 511)
                sidx[j, sl] = jnp.where(inr, dv - lo, tr)

    def fire_scat(sidx, gbuf):
        for j in range(ABLK):
            pltpu.async_copy(gbuf.at[pl.ds(j * LANE, LANE)],
                             acc.at[sidx.at[j]], sems, add=True)

    def wait_scat(sidx, gbuf):
        for j in range(ABLK):
            pltpu.make_async_copy(gbuf.at[pl.ds(j * LANE, LANE)],
                                  acc.at[sidx.at[j]], sems).wait()

    pltpu.sync_copy(reff_hbm.at[pl.ds(row0, ABLK)], ridx0)
    pltpu.sync_copy(dst_hbm.at[pl.ds(row0, ABLK)], didx0)
    fire_gather(ridx0, gbuf0)
    fire_idx(jnp.int32(1), ridx1, didx1)

    compute_sidx(jnp.int32(0), didx0, sidx0)
    wait_gather(gbuf0)
    wait_idx(ridx1, didx1)
    fire_gather(ridx1, gbuf1)
    fire_idx(jnp.int32(2), ridx0, didx0)
    fire_scat(sidx0, gbuf0)
    compute_sidx(jnp.int32(1), didx1, sidx1)
    wait_gather(gbuf1)
    wait_idx(ridx0, didx0)
    wait_scat(sidx0, gbuf0)
    fire_gather(ridx0, gbuf0)
    fire_idx(jnp.int32(3), ridx1, didx1)
    fire_scat(sidx1, gbuf1)

    def group(g, carry):
        b0 = 2 * g
        compute_sidx(b0, didx0, sidx0)
        wait_gather(gbuf0)
        wait_idx(ridx1, didx1)
        wait_scat(sidx1, gbuf1)
        fire_gather(ridx1, gbuf1)
        fire_idx(b0 + 2, ridx0, didx0)
        fire_scat(sidx0, gbuf0)
        compute_sidx(b0 + 1, didx1, sidx1)
        wait_gather(gbuf1)
        wait_idx(ridx0, didx0)
        wait_scat(sidx0, gbuf0)
        fire_gather(ridx0, gbuf0)
        fire_idx(b0 + 3, ridx1, didx1)
        fire_scat(sidx1, gbuf1)
        return carry

    lax.fori_loop(1, NBLK // 2, group, 0)
    wait_gather(gbuf0)
    wait_idx(ridx1, didx1)
    wait_scat(sidx1, gbuf1)

    plsc.subcore_barrier()
    wrow0 = s * (HALFN // NS)
    for t in range(2):
        r0 = wrow0 + t * LANE
        pltpu.sync_copy(acc.at[pl.ds(r0, LANE)], stage)
        pltpu.sync_copy(stage, agg_hbm.at[pl.ds(c * HALFN + r0, LANE)])
    r0 = wrow0 + 2 * LANE
    pltpu.sync_copy(acc.at[pl.ds(r0, 64)], stage.at[pl.ds(0, 64)])
    pltpu.sync_copy(stage.at[pl.ds(0, 64)],
                    agg_hbm.at[pl.ds(c * HALFN + r0, 64)])


def _tc_scale_matmul(h_pad, w, dis_pad):
    def body(h_ref, w_ref, dis_ref, out_ref):
        out_ref[...] = jnp.dot(h_ref[...], w_ref[...],
                               preferred_element_type=jnp.float32) * dis_ref[...]

    return pl.pallas_call(
        body,
        grid=(P // 512,),
        in_specs=[
            pl.BlockSpec((512, 128), lambda i: (i, 0)),
            pl.BlockSpec((128, 128), lambda i: (0, 0)),
            pl.BlockSpec((512, 1), lambda i: (i, 0)),
        ],
        out_specs=pl.BlockSpec((512, 128), lambda i: (i, 0)),
        out_shape=jax.ShapeDtypeStruct((P, 128), jnp.float32),
    )(h_pad, w, dis_pad)


def _tc_epilogue(aggp, ytab, dis_pad, b, g, lb, w_next):
    def body(agg_ref, ytab_ref, dis_ref, b_ref, g_ref, lb_ref, *rest):
        if w_next is not None:
            w_ref, out_ref = rest
        else:
            (out_ref,) = rest
        a = agg_ref[...]
        dis = dis_ref[...]
        o = dis * (a + ALPHA * ytab_ref[...]) + b_ref[...]
        h = jnp.maximum(o, 0.0)
        mu = jnp.mean(h, axis=-1, keepdims=True)
        var = jnp.mean((h - mu) ** 2, axis=-1, keepdims=True)
        hn = (h - mu) * lax.rsqrt(var + 1e-5) * g_ref[...] + lb_ref[...]
        if w_next is not None:
            out_ref[...] = jnp.dot(hn, w_ref[...],
                                   preferred_element_type=jnp.float32) * dis
        else:
            out_ref[...] = hn * dis

    in_specs = [
        pl.BlockSpec((512, 128), lambda i: (i, 0)),
        pl.BlockSpec((512, 128), lambda i: (i, 0)),
        pl.BlockSpec((512, 1), lambda i: (i, 0)),
        pl.BlockSpec((1, 128), lambda i: (0, 0)),
        pl.BlockSpec((1, 128), lambda i: (0, 0)),
        pl.BlockSpec((1, 128), lambda i: (0, 0)),
    ]
    args = [aggp, ytab, dis_pad, b, g, lb]
    if w_next is not None:
        in_specs.append(pl.BlockSpec((128, 128), lambda i: (0, 0)))
        args.append(w_next)
    return pl.pallas_call(
        body,
        grid=(P // 512,),
        in_specs=in_specs,
        out_specs=pl.BlockSpec((512, 128), lambda i: (i, 0)),
        out_shape=jax.ShapeDtypeStruct((P, 128), jnp.float32),
    )(*args)


def _tc_final(aggp, ttab, dis_pad, w2, b2):
    def body(agg_ref, t_ref, dis_ref, w_ref, b_ref, emb_ref, logp_ref):
        a = agg_ref[...]
        pre = dis_ref[...] * (a + ALPHA * t_ref[...])
        emb = jnp.dot(pre, w_ref[...],
                      preferred_element_type=jnp.float32) + b_ref[...]
        m = jnp.max(emb, axis=-1, keepdims=True)
        ex = jnp.exp(emb - m)
        lse = jnp.log(jnp.sum(ex, axis=-1, keepdims=True)) + m
        emb_ref[...] = emb
        logp_ref[...] = emb - lse

    return pl.pallas_call(
        body,
        grid=(25,),
        in_specs=[
            pl.BlockSpec((400, 128), lambda i: (i, 0)),
            pl.BlockSpec((400, 128), lambda i: (i, 0)),
            pl.BlockSpec((400, 1), lambda i: (i, 0)),
            pl.BlockSpec((128, 40), lambda i: (0, 0)),
            pl.BlockSpec((1, 40), lambda i: (0, 0)),
        ],
        out_specs=[
            pl.BlockSpec((400, 40), lambda i: (i, 0)),
            pl.BlockSpec((400, 40), lambda i: (i, 0)),
        ],
        out_shape=[
            jax.ShapeDtypeStruct((N, 40), jnp.float32),
            jax.ShapeDtypeStruct((N, 40), jnp.float32),
        ],
    )(aggp, ttab, dis_pad, w2, b2)


def kernel(x, edge_index, W0, b0, W1, b1, W2, b2, ln0_g, ln0_b, ln1_g, ln1_b):
    src = edge_index[0].astype(jnp.int32)
    dst = edge_index[1].astype(jnp.int32)
    pad_len = EP - ES
    pad_node = (jnp.arange(pad_len, dtype=jnp.int32)) % N
    s_all = jnp.concatenate([src, dst, pad_node])
    d_all = jnp.concatenate([dst, src, pad_node])
    keys = s_all * N + d_all

    keys2 = keys.reshape(EROWS, LANE)
    s2 = s_all.reshape(EROWS, LANE)
    d2 = d_all.reshape(EROWS, LANE)

    t_tab = _sc_dedup_scatter(keys2)
    reff2, deg_p = _sc_dedup_check(keys2, s2, d2, t_tab)

    deg = deg_p[:N] + deg_p[N:] + jnp.float32(ALPHA)
    dis = deg ** -0.5
    dis_pad = jnp.pad(dis, (0, P - N)).reshape(P, 1)

    x_pad = jnp.pad(x, ((0, P - N), (0, 0)))

    y0 = _tc_scale_matmul(x_pad, W0, dis_pad)
    agg0 = _sc_aggregate(y0, reff2, d2)
    y1 = _tc_epilogue(agg0, y0, dis_pad, b0.reshape(1, 128),
                      ln0_g.reshape(1, 128), ln0_b.reshape(1, 128), W1)
    agg1 = _sc_aggregate(y1, reff2, d2)
    t2 = _tc_epilogue(agg1, y1, dis_pad, b1.reshape(1, 128),
                      ln1_g.reshape(1, 128), ln1_b.reshape(1, 128), None)
    agg2 = _sc_aggregate(t2, reff2, d2)
    emb, logp = _tc_final(agg2, t2, dis_pad, W2, b2.reshape(1, 40))
    return (emb, logp)
